# baseline (speedup 1.0000x reference)
import functools

import jax
import jax.numpy as jnp
from jax import lax
from jax.experimental import pallas as pl
from jax.experimental.pallas import tpu as pltpu
from jax.experimental.pallas import tpu_sc as plsc

BATCH = 16384
D = 16
NC = 2
NS = 16
NW = NC * NS
BPW = BATCH // NW
NJ = BPW // 128
L = 16
NE = BPW * D


def _sc_body(uid_hbm, pid_hbm, nid_hbm, uemb_hbm, iemb_hbm, ubias_hbm,
             ibias_hbm, gb_hbm, pos_out, neg_out,
             uid_v, pid_v, nid_v, uix_v, pix_v, nix_v, ug_v, ipg_v, ing_v,
             ub_v, ibp_v, ibn_v, gb_v, outp_v, outn_v, sem):
    wid = lax.axis_index("s") * NC + lax.axis_index("c")
    base = wid * BPW

    s0 = wid // 2
    s1 = (wid % 2) * NJ
    pltpu.sync_copy(uid_hbm.at[s0, pl.ds(s1, NJ)], uid_v)
    pltpu.sync_copy(pid_hbm.at[s0, pl.ds(s1, NJ)], pid_v)
    pltpu.sync_copy(nid_hbm.at[s0, pl.ds(s1, NJ)], nid_v)
    pltpu.sync_copy(gb_hbm, gb_v)

    for idv, ixv in ((uid_v, uix_v), (pid_v, pix_v), (nid_v, nix_v)):
        for j in range(NJ):
            for c in range(128 // L):
                idd = idv[j, pl.ds(c * L, L)] * D
                for d in range(D):
                    ixv[pl.ds(j * (D * 128) + d * 128 + c * L, L)] = idd + d

    copies = []
    for j in range(NJ):
        for d in range(D):
            r = pl.ds(j * (D * 128) + d * 128, 128)
            copies.append(pltpu.async_copy(uemb_hbm.at[uix_v.at[r]], ug_v.at[r], sem))
            copies.append(pltpu.async_copy(iemb_hbm.at[pix_v.at[r]], ipg_v.at[r], sem))
            copies.append(pltpu.async_copy(iemb_hbm.at[nix_v.at[r]], ing_v.at[r], sem))
    for j in range(NJ):
        rows = pl.ds(j * 128, 128)
        copies.append(pltpu.async_copy(ubias_hbm.at[uid_v.at[j]], ub_v.at[rows], sem))
        copies.append(pltpu.async_copy(ibias_hbm.at[pid_v.at[j]], ibp_v.at[rows], sem))
        copies.append(pltpu.async_copy(ibias_hbm.at[nid_v.at[j]], ibn_v.at[rows], sem))
    for c in copies:
        c.wait()

    gb = gb_v[...]
    for j in range(NJ):
        for c in range(128 // L):
            e = pl.ds(j * 128 + c * L, L)
            accp = ub_v[e] + ibp_v[e] + gb
            accn = ub_v[e] + ibn_v[e] + gb
            for d in range(D):
                s = pl.ds(j * (D * 128) + d * 128 + c * L, L)
                u = ug_v[s]
                accp = accp + u * ipg_v[s]
                accn = accn + u * ing_v[s]
            outp_v[e] = accp
            outn_v[e] = accn

    pltpu.sync_copy(outp_v, pos_out.at[pl.ds(base, BPW)])
    pltpu.sync_copy(outn_v, neg_out.at[pl.ds(base, BPW)])


_sc_call = pl.kernel(
    _sc_body,
    out_type=(
        jax.ShapeDtypeStruct((BATCH,), jnp.float32),
        jax.ShapeDtypeStruct((BATCH,), jnp.float32),
    ),
    mesh=plsc.VectorSubcoreMesh(core_axis_name="c", subcore_axis_name="s",
                                num_cores=NC, num_subcores=NS),
    compiler_params=pltpu.CompilerParams(needs_layout_passes=False,
                                         use_tc_tiling_on_sc=False),
    scratch_types=[
        pltpu.VMEM((NJ, 128), jnp.int32),
        pltpu.VMEM((NJ, 128), jnp.int32),
        pltpu.VMEM((NJ, 128), jnp.int32),
        pltpu.VMEM((NE,), jnp.int32),
        pltpu.VMEM((NE,), jnp.int32),
        pltpu.VMEM((NE,), jnp.int32),
        pltpu.VMEM((NE,), jnp.float32),
        pltpu.VMEM((NE,), jnp.float32),
        pltpu.VMEM((NE,), jnp.float32),
        pltpu.VMEM((BPW,), jnp.float32),
        pltpu.VMEM((BPW,), jnp.float32),
        pltpu.VMEM((BPW,), jnp.float32),
        pltpu.VMEM((L,), jnp.float32),
        pltpu.VMEM((BPW,), jnp.float32),
        pltpu.VMEM((BPW,), jnp.float32),
        pltpu.SemaphoreType.DMA,
    ],
)


def kernel(user_ids, pos_item_ids, neg_item_ids, user_emb, item_emb,
           user_bias, item_bias, global_bias):
    uid = user_ids.astype(jnp.int32).reshape(BATCH // 1024, 8, 128)
    pid = pos_item_ids.astype(jnp.int32).reshape(BATCH // 1024, 8, 128)
    nid = neg_item_ids.astype(jnp.int32).reshape(BATCH // 1024, 8, 128)
    uef = user_emb.reshape(-1)
    ief = item_emb.reshape(-1)
    ubf = user_bias.reshape(-1)
    ibf = item_bias.reshape(-1)
    gb = jnp.broadcast_to(global_bias, (L,))
    return _sc_call(uid, pid, nid, uef, ief, ubf, ibf, gb)


# speedup vs baseline: 1.0491x; 1.0491x over previous
import functools

import jax
import jax.numpy as jnp
from jax import lax
from jax.experimental import pallas as pl
from jax.experimental.pallas import tpu as pltpu
from jax.experimental.pallas import tpu_sc as plsc

BATCH = 16384
EMBED_DIM = 16
NC = 2
NS = 16
NW = NC * NS
BPW = BATCH // NW
IDXC = 128
NJ = BPW // IDXC
L = 16
UNROLL = 8


def _sc_body(uid_hbm, pid_hbm, nid_hbm, uemb_hbm, iemb_hbm, ubias_hbm,
             ibias_hbm, gb_hbm, pos_out, neg_out,
             uid_v, pid_v, nid_v, ue_v, iep_v, ien_v, ub_v, ibp_v, ibn_v,
             gb_v, outp_v, outn_v, sem):
    wid = lax.axis_index("s") * NC + lax.axis_index("c")
    base = wid * BPW

    s0 = wid // 2
    s1 = (wid % 2) * NJ
    pltpu.sync_copy(uid_hbm.at[s0, pl.ds(s1, NJ)], uid_v)
    pltpu.sync_copy(pid_hbm.at[s0, pl.ds(s1, NJ)], pid_v)
    pltpu.sync_copy(nid_hbm.at[s0, pl.ds(s1, NJ)], nid_v)
    pltpu.sync_copy(gb_hbm, gb_v)

    copies = []
    for j in range(NJ):
        rows = pl.ds(j * IDXC, IDXC)
        copies.append(pltpu.async_copy(uemb_hbm.at[uid_v.at[j]], ue_v.at[rows], sem))
        copies.append(pltpu.async_copy(iemb_hbm.at[pid_v.at[j]], iep_v.at[rows], sem))
        copies.append(pltpu.async_copy(iemb_hbm.at[nid_v.at[j]], ien_v.at[rows], sem))
        copies.append(pltpu.async_copy(ubias_hbm.at[uid_v.at[j]], ub_v.at[rows], sem))
        copies.append(pltpu.async_copy(ibias_hbm.at[pid_v.at[j]], ibp_v.at[rows], sem))
        copies.append(pltpu.async_copy(ibias_hbm.at[nid_v.at[j]], ibn_v.at[rows], sem))
    for c in copies:
        c.wait()

    def dot_body(i, carry):
        for k in range(UNROLL):
            e = i * UNROLL + k
            ue = ue_v[e]
            iep_v[e] = plsc.cumsum(ue * iep_v[e])
            ien_v[e] = plsc.cumsum(ue * ien_v[e])
        return carry

    lax.fori_loop(0, BPW // UNROLL, dot_body, 0)

    lane = lax.iota(jnp.int32, L)
    col15 = jnp.full((L,), EMBED_DIM - 1, jnp.int32)
    gb = gb_v[...]
    for c in range(BPW // L):
        s = pl.ds(c * L, L)
        rows = lane + (c * L)
        dp = plsc.load_gather(iep_v, [rows, col15])
        dn = plsc.load_gather(ien_v, [rows, col15])
        ub = ub_v[s] + gb
        outp_v[s] = dp + (ub + ibp_v[s])
        outn_v[s] = dn + (ub + ibn_v[s])

    pltpu.sync_copy(outp_v, pos_out.at[pl.ds(base, BPW)])
    pltpu.sync_copy(outn_v, neg_out.at[pl.ds(base, BPW)])


_sc_call = pl.kernel(
    _sc_body,
    out_type=(
        jax.ShapeDtypeStruct((BATCH,), jnp.float32),
        jax.ShapeDtypeStruct((BATCH,), jnp.float32),
    ),
    mesh=plsc.VectorSubcoreMesh(core_axis_name="c", subcore_axis_name="s",
                                num_cores=NC, num_subcores=NS),
    compiler_params=pltpu.CompilerParams(needs_layout_passes=False,
                                         use_tc_tiling_on_sc=False),
    scratch_types=[
        pltpu.VMEM((NJ, IDXC), jnp.int32),
        pltpu.VMEM((NJ, IDXC), jnp.int32),
        pltpu.VMEM((NJ, IDXC), jnp.int32),
        pltpu.VMEM((BPW, EMBED_DIM), jnp.float32),
        pltpu.VMEM((BPW, EMBED_DIM), jnp.float32),
        pltpu.VMEM((BPW, EMBED_DIM), jnp.float32),
        pltpu.VMEM((BPW,), jnp.float32),
        pltpu.VMEM((BPW,), jnp.float32),
        pltpu.VMEM((BPW,), jnp.float32),
        pltpu.VMEM((L,), jnp.float32),
        pltpu.VMEM((BPW,), jnp.float32),
        pltpu.VMEM((BPW,), jnp.float32),
        pltpu.SemaphoreType.DMA,
    ],
)


def kernel(user_ids, pos_item_ids, neg_item_ids, user_emb, item_emb,
           user_bias, item_bias, global_bias):
    uid = user_ids.astype(jnp.int32).reshape(BATCH // 1024, 8, 128)
    pid = pos_item_ids.astype(jnp.int32).reshape(BATCH // 1024, 8, 128)
    nid = neg_item_ids.astype(jnp.int32).reshape(BATCH // 1024, 8, 128)
    ub = jnp.sum(user_bias, axis=1)
    ib = jnp.sum(item_bias, axis=1)
    gb = jnp.broadcast_to(global_bias, (L,))
    return _sc_call(uid, pid, nid, user_emb, item_emb, ub, ib, gb)
